# baseline (speedup 1.0000x reference)
import functools

import jax
import jax.numpy as jnp
from jax import lax
from jax.experimental import pallas as pl
from jax.experimental.pallas import tpu as pltpu
from jax.experimental.pallas import tpu_sc as plsc

N = 4608
C = 768
E = 5
TILES = 32
TPT = N // TILES
BLK = 256
MAXB = (2 * N) // BLK + E - 1
CAP = MAXB * BLK
BT = 512
NEG = -1e30


def _router_body(x_ref, wgt_ref, idx_ref, w_ref):
    x = x_ref[...]
    wgt = wgt_ref[...]
    lT = lax.dot_general(wgt, x, (((1,), (1,)), ((), ())),
                         precision=lax.Precision.HIGHEST,
                         preferred_element_type=jnp.float32)
    r_ids = lax.broadcasted_iota(jnp.int32, lT.shape, 0)
    lT = jnp.where(r_ids < E, lT, NEG)
    big = jnp.int32(8)
    m1 = jnp.max(lT, axis=0, keepdims=True)
    idx1 = jnp.min(jnp.where(lT == m1, r_ids, big), axis=0, keepdims=True)
    l2 = jnp.where(r_ids == idx1, NEG, lT)
    m2 = jnp.max(l2, axis=0, keepdims=True)
    idx2 = jnp.min(jnp.where(l2 == m2, r_ids, big), axis=0, keepdims=True)
    w1 = 1.0 / (1.0 + jnp.exp(m2 - m1))
    zi = jnp.zeros((6, BT), jnp.int32)
    zf = zi.astype(jnp.float32)
    idx_ref[...] = jnp.concatenate([idx1, idx2, zi], axis=0)
    w_ref[...] = jnp.concatenate([w1, 1.0 - w1, zf], axis=0)


def _router(xf, WgT8):
    return pl.pallas_call(
        _router_body,
        grid=(N // BT,),
        in_specs=[
            pl.BlockSpec((BT, C), lambda j: (j, 0)),
            pl.BlockSpec((8, C), lambda j: (0, 0)),
        ],
        out_specs=[
            pl.BlockSpec((8, BT), lambda j: (0, j)),
            pl.BlockSpec((8, BT), lambda j: (0, j)),
        ],
        out_shape=[
            jax.ShapeDtypeStruct((8, N), jnp.int32),
            jax.ShapeDtypeStruct((8, N), jnp.float32),
        ],
        compiler_params=pltpu.CompilerParams(
            dimension_semantics=("arbitrary",)),
    )(xf, WgT8)


_MESH = plsc.VectorSubcoreMesh(core_axis_name="c", subcore_axis_name="s",
                               num_cores=2, num_subcores=16)
NCHUNK = TPT // 16
GROUPS = (2 * TPT) // 48
ALLCH = N // 16


def _splat(v):
    return jnp.broadcast_to(v, (16,))


def _k2_body(idx_flat, x_hbm, xs_out, pos_out, bexp_out,
             e1f, e2f, destflat, src2d, dst2d, rowbuf, bexp_buf, sem):
    wid = lax.axis_index("s") * 2 + lax.axis_index("c")
    t0 = wid * TPT
    pltpu.sync_copy(idx_flat.at[pl.ds(0, N)], e1f)
    pltpu.sync_copy(idx_flat.at[pl.ds(N, N)], e2f)
    lanes = lax.iota(jnp.int32, 16)
    zero5 = tuple(jnp.zeros((16,), jnp.int32) for _ in range(E))

    def cnt_chunk(c, acc):
        v1 = e1f[pl.ds(16 * c, 16)]
        v2 = e2f[pl.ds(16 * c, 16)]
        return tuple(acc[e] + (v1 == e).astype(jnp.int32)
                     + (v2 == e).astype(jnp.int32) for e in range(E))

    tot_acc = lax.fori_loop(0, ALLCH, cnt_chunk, zero5)
    pre_acc = lax.fori_loop(0, wid * NCHUNK, cnt_chunk, zero5)
    tot = [jnp.sum(tot_acc[e]) for e in range(E)]
    pre = [jnp.sum(pre_acc[e]) for e in range(E)]
    nb = [(t + (BLK - 1)) // BLK for t in tot]
    base = []
    b = jnp.int32(0)
    for e in range(E):
        base.append(b)
        b = b + nb[e] * BLK
    cum = []
    cb = jnp.int32(0)
    for e in range(E):
        cb = cb + nb[e]
        cum.append(cb)

    @pl.when(wid == 0)
    def _():
        for k in range(3):
            bid = lanes + 16 * k
            v = jnp.zeros((16,), jnp.int32)
            for e in range(1, E):
                v = jnp.where(bid >= cum[e - 1], e, v)
            v = jnp.where(bid >= cum[E - 1], 0, v)
            bexp_buf[pl.ds(16 * k, 16)] = v
        pltpu.sync_copy(bexp_buf, bexp_out)

    run = [base[e] + pre[e] for e in range(E)]
    for slot in range(2):
        ef = e1f if slot == 0 else e2f
        for k in range(NCHUNK):
            ev = ef[pl.ds(t0 + 16 * k, 16)]
            dest = jnp.zeros((16,), jnp.int32)
            for e in range(E):
                mask = ev == e
                mi = mask.astype(jnp.int32)
                incl = plsc.cumsum(mi)
                dest = jnp.where(mask, run[e] + incl - 1, dest)
                run[e] = run[e] + jnp.sum(mi)
            a0 = slot * TPT + 16 * k
            destflat[pl.ds(a0, 16)] = dest
            g, off = a0 // 48, a0 % 48
            dst2d[g, pl.ds(off, 16)] = dest
            src2d[g, pl.ds(off, 16)] = t0 + 16 * k + lanes
    pltpu.sync_copy(destflat.at[pl.ds(0, TPT)], pos_out.at[pl.ds(t0, TPT)])
    pltpu.sync_copy(destflat.at[pl.ds(TPT, TPT)],
                    pos_out.at[pl.ds(N + t0, TPT)])
    for g in range(GROUPS):
        pltpu.async_copy(x_hbm.at[src2d.at[g]], rowbuf, sem).wait()
        pltpu.async_copy(rowbuf, xs_out.at[dst2d.at[g]], sem).wait()


def _route_sc(idx_flat, xf):
    k2 = functools.partial(
        pl.kernel,
        out_type=[
            jax.ShapeDtypeStruct((CAP, C), jnp.float32),
            jax.ShapeDtypeStruct((2 * N,), jnp.int32),
            jax.ShapeDtypeStruct((48,), jnp.int32),
        ],
        mesh=_MESH,
        scratch_types=[
            pltpu.VMEM((N,), jnp.int32),
            pltpu.VMEM((N,), jnp.int32),
            pltpu.VMEM((2 * TPT,), jnp.int32),
            pltpu.VMEM((GROUPS, 48), jnp.int32),
            pltpu.VMEM((GROUPS, 48), jnp.int32),
            pltpu.VMEM((48, C), jnp.float32),
            pltpu.VMEM((48,), jnp.int32),
            pltpu.SemaphoreType.DMA,
        ],
        compiler_params=pltpu.CompilerParams(needs_layout_passes=False),
    )(_k2_body)
    return k2(idx_flat, xf)


def _gmm_body(bexp_ref, xs_ref, we_ref, be_ref, y_ref):
    del bexp_ref
    y_ref[...] = (jnp.dot(xs_ref[...], we_ref[0],
                          precision=lax.Precision.HIGHEST,
                          preferred_element_type=jnp.float32)
                  + be_ref[0])


def _gmm(bexp, xs, We, be):
    grid_spec = pltpu.PrefetchScalarGridSpec(
        num_scalar_prefetch=1,
        grid=(MAXB,),
        in_specs=[
            pl.BlockSpec((BLK, C), lambda i, b: (i, 0)),
            pl.BlockSpec((1, C, C), lambda i, b: (b[i], 0, 0)),
            pl.BlockSpec((1, 1, C), lambda i, b: (b[i], 0, 0)),
        ],
        out_specs=pl.BlockSpec((BLK, C), lambda i, b: (i, 0)),
    )
    return pl.pallas_call(
        _gmm_body,
        grid_spec=grid_spec,
        out_shape=jax.ShapeDtypeStruct((CAP, C), jnp.float32),
        compiler_params=pltpu.CompilerParams(
            dimension_semantics=("arbitrary",)),
    )(bexp, xs, We, be.reshape(E, 1, C))


TG = 3
NV = C // 16


def _k4_body(y_hbm, pos, wts, out_hbm, p0, p1, w0, w1, yb0, yb1, ob, sem):
    wid = lax.axis_index("s") * 2 + lax.axis_index("c")
    t0 = wid * TPT
    pltpu.sync_copy(pos.at[pl.ds(t0, TPT)], p0)
    pltpu.sync_copy(pos.at[pl.ds(N + t0, TPT)], p1)
    pltpu.sync_copy(wts.at[pl.ds(t0, TPT)], w0)
    pltpu.sync_copy(wts.at[pl.ds(N + t0, TPT)], w1)
    for g in range(TG):
        pltpu.async_copy(y_hbm.at[p0.at[pl.ds(48 * g, 48)]], yb0, sem).wait()
        pltpu.async_copy(y_hbm.at[p1.at[pl.ds(48 * g, 48)]], yb1, sem).wait()

        def body(t, carry):
            a = _splat(48 * g + t)
            wa = plsc.load_gather(w0, [a])
            wb = plsc.load_gather(w1, [a])
            for v in range(NV):
                acc = (wa * yb0[t, pl.ds(16 * v, 16)]
                       + wb * yb1[t, pl.ds(16 * v, 16)])
                ob[t, pl.ds(16 * v, 16)] = jnp.where(acc >= 0, acc, 0.01 * acc)
            return carry

        lax.fori_loop(0, 48, body, 0)
        pltpu.sync_copy(ob, out_hbm.at[pl.ds(t0 + 48 * g, 48)])


def _combine_sc(y, pos, wts):
    k4 = functools.partial(
        pl.kernel,
        out_type=jax.ShapeDtypeStruct((N, C), jnp.float32),
        mesh=_MESH,
        scratch_types=[
            pltpu.VMEM((TPT,), jnp.int32),
            pltpu.VMEM((TPT,), jnp.int32),
            pltpu.VMEM((TPT,), jnp.float32),
            pltpu.VMEM((TPT,), jnp.float32),
            pltpu.VMEM((48, C), jnp.float32),
            pltpu.VMEM((48, C), jnp.float32),
            pltpu.VMEM((48, C), jnp.float32),
            pltpu.SemaphoreType.DMA,
        ],
        compiler_params=pltpu.CompilerParams(needs_layout_passes=False),
    )(_k4_body)
    return k4(y, pos, wts)


@jax.jit
def kernel(x, Wg, We, be):
    B, H, W, Cc = x.shape
    xf = x.reshape(N, C)
    WgT8 = jnp.zeros((8, C), jnp.float32).at[:E].set(Wg.T)
    idxT, wT = _router(xf, WgT8)
    idx_flat = idxT[0:2].reshape(2 * N)
    w_flat = wT[0:2].reshape(2 * N)
    xs, pos, bexp = _route_sc(idx_flat, xf)
    y = _gmm(bexp[:MAXB], xs, We, be)
    out = _combine_sc(y, pos, w_flat)
    return out.reshape(B, H, W, Cc)


# speedup vs baseline: 1.2867x; 1.2867x over previous
import functools

import jax
import jax.numpy as jnp
from jax import lax
from jax.experimental import pallas as pl
from jax.experimental.pallas import tpu as pltpu
from jax.experimental.pallas import tpu_sc as plsc

N = 4608
C = 768
E = 5
TILES = 32
TPT = N // TILES
BLK = 256
MAXB = (2 * N) // BLK + E - 1
CAP = MAXB * BLK
BT = 512
NEG = -1e30


def _router_body(x_ref, wg_ref, idx_ref, w_ref):
    x = x_ref[...]
    wg = wg_ref[...]
    l = jnp.dot(x, wg, preferred_element_type=jnp.float32)
    e_ids = lax.broadcasted_iota(jnp.int32, l.shape, 1)
    l = jnp.where(e_ids < E, l, NEG)
    big = jnp.int32(8)
    m1 = jnp.max(l, axis=1, keepdims=True)
    idx1 = jnp.min(jnp.where(l == m1, e_ids, big), axis=1, keepdims=True)
    l2 = jnp.where(e_ids == idx1, NEG, l)
    m2 = jnp.max(l2, axis=1, keepdims=True)
    idx2 = jnp.min(jnp.where(l2 == m2, e_ids, big), axis=1, keepdims=True)
    w1 = 1.0 / (1.0 + jnp.exp(m2 - m1))
    zi = jnp.zeros((BT, 6), jnp.int32)
    zf = zi.astype(jnp.float32)
    idx_ref[...] = jnp.concatenate([idx1, idx2, zi], axis=1)
    w_ref[...] = jnp.concatenate([w1, 1.0 - w1, zf], axis=1)


def _router(xf, Wg8):
    return pl.pallas_call(
        _router_body,
        grid=(N // BT,),
        in_specs=[
            pl.BlockSpec((BT, C), lambda j: (j, 0)),
            pl.BlockSpec((C, 8), lambda j: (0, 0)),
        ],
        out_specs=[
            pl.BlockSpec((BT, 8), lambda j: (j, 0)),
            pl.BlockSpec((BT, 8), lambda j: (j, 0)),
        ],
        out_shape=[
            jax.ShapeDtypeStruct((N, 8), jnp.int32),
            jax.ShapeDtypeStruct((N, 8), jnp.float32),
        ],
        compiler_params=pltpu.CompilerParams(
            dimension_semantics=("arbitrary",)),
    )(xf, Wg8)


_MESH = plsc.VectorSubcoreMesh(core_axis_name="c", subcore_axis_name="s",
                               num_cores=2, num_subcores=16)
NCHUNK = TPT // 16
GROUPS = (2 * TPT) // 48
ALLCH = N // 16


def _splat(v):
    return jnp.broadcast_to(v, (16,))


def _k2_body(idx_flat, x_hbm, xs_out, pos_out, bexp_out,
             e1f, e2f, destflat, src2d, dst2d, rowbuf, bexp_buf, sem):
    wid = lax.axis_index("s") * 2 + lax.axis_index("c")
    t0 = wid * TPT
    pltpu.sync_copy(idx_flat.at[pl.ds(0, N)], e1f)
    pltpu.sync_copy(idx_flat.at[pl.ds(N, N)], e2f)
    lanes = lax.iota(jnp.int32, 16)
    zero5 = tuple(jnp.zeros((16,), jnp.int32) for _ in range(E))

    def cnt_chunk(c, acc):
        v1 = e1f[pl.ds(16 * c, 16)]
        v2 = e2f[pl.ds(16 * c, 16)]
        return tuple(acc[e] + (v1 == e).astype(jnp.int32)
                     + (v2 == e).astype(jnp.int32) for e in range(E))

    tot_acc = lax.fori_loop(0, ALLCH, cnt_chunk, zero5)
    pre_acc = lax.fori_loop(0, wid * NCHUNK, cnt_chunk, zero5)
    tot = [jnp.sum(tot_acc[e]) for e in range(E)]
    pre = [jnp.sum(pre_acc[e]) for e in range(E)]
    nb = [(t + (BLK - 1)) // BLK for t in tot]
    base = []
    b = jnp.int32(0)
    for e in range(E):
        base.append(b)
        b = b + nb[e] * BLK
    cum = []
    cb = jnp.int32(0)
    for e in range(E):
        cb = cb + nb[e]
        cum.append(cb)

    @pl.when(wid == 0)
    def _():
        for k in range(3):
            bid = lanes + 16 * k
            v = jnp.zeros((16,), jnp.int32)
            for e in range(1, E):
                v = jnp.where(bid >= cum[e - 1], e, v)
            v = jnp.where(bid >= cum[E - 1], 0, v)
            bexp_buf[pl.ds(16 * k, 16)] = v
        pltpu.sync_copy(bexp_buf, bexp_out)

    run = [base[e] + pre[e] for e in range(E)]
    for slot in range(2):
        ef = e1f if slot == 0 else e2f
        for k in range(NCHUNK):
            ev = ef[pl.ds(t0 + 16 * k, 16)]
            dest = jnp.zeros((16,), jnp.int32)
            for e in range(E):
                mask = ev == e
                mi = mask.astype(jnp.int32)
                incl = plsc.cumsum(mi)
                dest = jnp.where(mask, run[e] + incl - 1, dest)
                run[e] = run[e] + jnp.sum(mi)
            a0 = slot * TPT + 16 * k
            destflat[pl.ds(a0, 16)] = dest
            g, off = a0 // 48, a0 % 48
            dst2d[g, pl.ds(off, 16)] = dest
            src2d[g, pl.ds(off, 16)] = t0 + 16 * k + lanes
    pltpu.sync_copy(destflat.at[pl.ds(0, TPT)], pos_out.at[pl.ds(t0, TPT)])
    pltpu.sync_copy(destflat.at[pl.ds(TPT, TPT)],
                    pos_out.at[pl.ds(N + t0, TPT)])
    for g in range(GROUPS):
        pltpu.async_copy(x_hbm.at[src2d.at[g]], rowbuf, sem).wait()
        pltpu.async_copy(rowbuf, xs_out.at[dst2d.at[g]], sem).wait()


def _route_sc(idx_flat, xf):
    k2 = functools.partial(
        pl.kernel,
        out_type=[
            jax.ShapeDtypeStruct((CAP, C), jnp.float32),
            jax.ShapeDtypeStruct((2 * N,), jnp.int32),
            jax.ShapeDtypeStruct((48,), jnp.int32),
        ],
        mesh=_MESH,
        scratch_types=[
            pltpu.VMEM((N,), jnp.int32),
            pltpu.VMEM((N,), jnp.int32),
            pltpu.VMEM((2 * TPT,), jnp.int32),
            pltpu.VMEM((GROUPS, 48), jnp.int32),
            pltpu.VMEM((GROUPS, 48), jnp.int32),
            pltpu.VMEM((48, C), jnp.float32),
            pltpu.VMEM((48,), jnp.int32),
            pltpu.SemaphoreType.DMA,
        ],
        compiler_params=pltpu.CompilerParams(needs_layout_passes=False),
    )(_k2_body)
    return k2(idx_flat, xf)


def _gmm_body(bexp_ref, xs_ref, we_ref, be_ref, y_ref):
    del bexp_ref
    y_ref[...] = (jnp.dot(xs_ref[...], we_ref[0],
                          preferred_element_type=jnp.float32)
                  + be_ref[0])


def _gmm(bexp, xs, We, be):
    grid_spec = pltpu.PrefetchScalarGridSpec(
        num_scalar_prefetch=1,
        grid=(MAXB,),
        in_specs=[
            pl.BlockSpec((BLK, C), lambda i, b: (i, 0)),
            pl.BlockSpec((1, C, C), lambda i, b: (b[i], 0, 0)),
            pl.BlockSpec((1, 1, C), lambda i, b: (b[i], 0, 0)),
        ],
        out_specs=pl.BlockSpec((BLK, C), lambda i, b: (i, 0)),
    )
    return pl.pallas_call(
        _gmm_body,
        grid_spec=grid_spec,
        out_shape=jax.ShapeDtypeStruct((CAP, C), jnp.float32),
        compiler_params=pltpu.CompilerParams(
            dimension_semantics=("arbitrary",)),
    )(bexp, xs, We, be.reshape(E, 1, C))


TG = 3
NV = C // 16


def _k4_body(y_hbm, pos, wts, out_hbm, p0, p1, w0, w1, yb0, yb1, ob, sem):
    wid = lax.axis_index("s") * 2 + lax.axis_index("c")
    t0 = wid * TPT
    pltpu.sync_copy(pos.at[pl.ds(t0, TPT)], p0)
    pltpu.sync_copy(pos.at[pl.ds(N + t0, TPT)], p1)
    pltpu.sync_copy(wts.at[pl.ds(t0, TPT)], w0)
    pltpu.sync_copy(wts.at[pl.ds(N + t0, TPT)], w1)
    for g in range(TG):
        pltpu.async_copy(y_hbm.at[p0.at[pl.ds(48 * g, 48)]], yb0, sem).wait()
        pltpu.async_copy(y_hbm.at[p1.at[pl.ds(48 * g, 48)]], yb1, sem).wait()

        def body(t, carry):
            a = _splat(48 * g + t)
            wa = plsc.load_gather(w0, [a])
            wb = plsc.load_gather(w1, [a])
            for v in range(NV):
                acc = (wa * yb0[t, pl.ds(16 * v, 16)]
                       + wb * yb1[t, pl.ds(16 * v, 16)])
                ob[t, pl.ds(16 * v, 16)] = jnp.where(acc >= 0, acc, 0.01 * acc)
            return carry

        lax.fori_loop(0, 48, body, 0)
        pltpu.sync_copy(ob, out_hbm.at[pl.ds(t0 + 48 * g, 48)])


def _combine_sc(y, pos, wts):
    k4 = functools.partial(
        pl.kernel,
        out_type=jax.ShapeDtypeStruct((N, C), jnp.float32),
        mesh=_MESH,
        scratch_types=[
            pltpu.VMEM((TPT,), jnp.int32),
            pltpu.VMEM((TPT,), jnp.int32),
            pltpu.VMEM((TPT,), jnp.float32),
            pltpu.VMEM((TPT,), jnp.float32),
            pltpu.VMEM((48, C), jnp.float32),
            pltpu.VMEM((48, C), jnp.float32),
            pltpu.VMEM((48, C), jnp.float32),
            pltpu.SemaphoreType.DMA,
        ],
        compiler_params=pltpu.CompilerParams(needs_layout_passes=False),
    )(_k4_body)
    return k4(y, pos, wts)


@jax.jit
def kernel(x, Wg, We, be):
    B, H, W, Cc = x.shape
    xf = x.reshape(N, C)
    Wg8 = jnp.zeros((C, 8), jnp.float32).at[:, :E].set(Wg)
    idx_rc, w_rc = _router(xf, Wg8)
    idx_flat = jnp.concatenate([idx_rc[:, 0], idx_rc[:, 1]])
    w_flat = jnp.concatenate([w_rc[:, 0], w_rc[:, 1]])
    xs, pos, bexp = _route_sc(idx_flat, xf)
    y = _gmm(bexp[:MAXB], xs, We, be)
    out = _combine_sc(y, pos, w_flat)
    return out.reshape(B, H, W, Cc)


# speedup vs baseline: 4.5054x; 3.5016x over previous
import functools

import jax
import jax.numpy as jnp
from jax.experimental import pallas as pl
from jax.experimental.pallas import tpu as pltpu

DIM = 768
NUM_EXPERTS = 5
TOP_K = 2


def _moe_block(x_ref, wg_ref, we_ref, be_ref, out_ref):
    x = x_ref[...]
    wg = wg_ref[...]
    logits = jnp.dot(x, wg, preferred_element_type=jnp.float32)
    e_ids = jax.lax.broadcasted_iota(jnp.int32, logits.shape, 1)
    big = jnp.int32(NUM_EXPERTS)
    m1 = jnp.max(logits, axis=-1, keepdims=True)
    idx1 = jnp.min(jnp.where(logits == m1, e_ids, big), axis=-1, keepdims=True)
    mask1 = e_ids == idx1
    l2 = jnp.where(mask1, -jnp.inf, logits)
    m2 = jnp.max(l2, axis=-1, keepdims=True)
    idx2 = jnp.min(jnp.where(l2 == m2, e_ids, big), axis=-1, keepdims=True)
    mask2 = e_ids == idx2
    w1 = 1.0 / (1.0 + jnp.exp(m2 - m1))
    combine = jnp.where(mask1, w1, 0.0) + jnp.where(mask2, 1.0 - w1, 0.0)
    acc = jnp.dot(combine, be_ref[...], preferred_element_type=jnp.float32)
    for e in range(NUM_EXPERTS):
        y = jnp.dot(x, we_ref[e], preferred_element_type=jnp.float32)
        acc = acc + combine[:, e:e + 1] * y
    out_ref[...] = jnp.where(acc >= 0, acc, 0.01 * acc)


@functools.partial(jax.jit, static_argnames=())
def kernel(x, Wg, We, be):
    B, H, W, C = x.shape
    E = Wg.shape[1]
    N = B * H * W
    xf = x.reshape(N, C)
    BN = 768
    grid = (N // BN,)
    out = pl.pallas_call(
        _moe_block,
        grid=grid,
        in_specs=[
            pl.BlockSpec((BN, C), lambda i: (i, 0)),
            pl.BlockSpec((C, E), lambda i: (0, 0)),
            pl.BlockSpec((E, C, C), lambda i: (0, 0, 0)),
            pl.BlockSpec((E, C), lambda i: (0, 0)),
        ],
        out_specs=pl.BlockSpec((BN, C), lambda i: (i, 0)),
        out_shape=jax.ShapeDtypeStruct((N, C), jnp.float32),
        compiler_params=pltpu.CompilerParams(
            dimension_semantics=("arbitrary",),
        ),
    )(xf, Wg, We, be)
    return out.reshape(B, H, W, C)
